# baseline (speedup 1.0000x reference)
import functools

import jax
import jax.numpy as jnp
from jax import lax
from jax.experimental import pallas as pl
from jax.experimental.pallas import tpu as pltpu
from jax.experimental.pallas import tpu_sc as plsc

NC = 2
NS = 16
NW = NC * NS
L = 16
CH = 128


def _round_up(a, b):
    return (a + b - 1) // b * b


def _make_deg_kernel(pt, R):
    mesh = plsc.VectorSubcoreMesh(core_axis_name="c", subcore_axis_name="s",
                                  num_cores=NC, num_subcores=NS)

    @functools.partial(
        pl.kernel,
        out_type=jax.ShapeDtypeStruct((NW, R), jnp.float32),
        mesh=mesh,
        scratch_types=[
            pltpu.VMEM((pt,), jnp.int32),
            pltpu.VMEM((R,), jnp.float32),
        ],
        compiler_params=pltpu.CompilerParams(needs_layout_passes=False),
    )
    def deg_kernel(dst_hbm, out_hbm, dstbuf, degloc):
        c = lax.axis_index("c")
        s = lax.axis_index("s")
        wid = s * NC + c
        pltpu.sync_copy(dst_hbm.at[pl.ds(wid * pt, pt)], dstbuf)

        def zbody(i, carry):
            degloc[pl.ds(i * L, L)] = jnp.zeros((L,), jnp.float32)
            return carry

        lax.fori_loop(0, R // L, zbody, 0)

        ones = jnp.ones((L,), jnp.float32)

        def cbody(j, carry):
            idx = dstbuf[pl.ds(j * L, L)]
            plsc.addupdate_scatter(degloc, [idx], ones)
            return carry

        lax.fori_loop(0, pt // L, cbody, 0)
        pltpu.sync_copy(degloc, out_hbm.at[wid])

    return deg_kernel


def _make_scatter_kernel(pt, R, dh, nb):
    mesh = plsc.VectorSubcoreMesh(core_axis_name="c", subcore_axis_name="s",
                                  num_cores=NC, num_subcores=NS)
    rpt = R // NS
    n_full = pt // CH
    rem = pt % CH
    dg = 3

    scratch = [
        pltpu.VMEM((pt,), jnp.int32),
        pltpu.VMEM((pt,), jnp.int32),
        pltpu.VMEM((CH,), jnp.int32),
    ]
    scratch += [pltpu.VMEM((CH, dh), jnp.float32)] * nb
    scratch += [pltpu.VMEM((CH,), jnp.int32)] * nb
    if rem:
        scratch += [pltpu.VMEM((rem, dh), jnp.float32),
                    pltpu.VMEM((rem,), jnp.int32),
                    pltpu.VMEM((rem,), jnp.int32)]
    scratch += [pltpu.VMEM_SHARED((R, dh), jnp.float32)]
    scratch += [pltpu.SemaphoreType.DMA] * nb

    @functools.partial(
        pl.kernel,
        out_type=jax.ShapeDtypeStruct((NC, R, dh), jnp.float32),
        mesh=mesh,
        scratch_types=scratch,
        compiler_params=pltpu.CompilerParams(needs_layout_passes=False,
                                             use_tc_tiling_on_sc=False),
    )
    def scatter_kernel(hp_hbm, src_hbm, dst_hbm, selfsrc_hbm, selfdst_hbm,
                       zeros_hbm, out_hbm, srcbuf, dstbuf, didx, *rest):
        rows = rest[:nb]
        sidx = rest[nb:2 * nb]
        rest = rest[2 * nb:]
        if rem:
            rows_t, sidx_t, didx_t = rest[:3]
            rest = rest[3:]
        agg = rest[0]
        gsems = rest[1:]

        c = lax.axis_index("c")
        s = lax.axis_index("s")
        table = hp_hbm.at[c]
        pltpu.sync_copy(zeros_hbm, agg.at[pl.ds(s * rpt, rpt)])
        pltpu.sync_copy(src_hbm.at[pl.ds(s * pt, pt)], srcbuf)
        pltpu.sync_copy(dst_hbm.at[pl.ds(s * pt, pt)], dstbuf)
        plsc.subcore_barrier()

        def fire_gather(j, b):
            for k in range(CH // L):
                sidx[b][pl.ds(k * L, L)] = srcbuf[pl.ds(j * CH + k * L, L)]
            pltpu.async_copy(table.at[sidx[b]], rows[b], gsems[b])

        def finish_chunk(j, b):
            pltpu.make_async_copy(
                table.at[sidx[b]], rows[b], gsems[b]).wait()
            for k in range(CH // L):
                didx[pl.ds(k * L, L)] = dstbuf[pl.ds(j * CH + k * L, L)]
            pltpu.sync_copy(rows[b], agg.at[didx], add=True)

        for k in range(dg):
            fire_gather(k, k)

        def body(g, carry):
            for b in range(nb):
                j = g * nb + b
                finish_chunk(j, b)
                fire_gather(j + dg, (b + dg) % nb)
            return carry

        n_main = (n_full - nb) // nb
        lax.fori_loop(0, n_main, body, 0)
        for j in range(n_main * nb, n_full):
            b = j % nb
            finish_chunk(j, b)
            if j + dg < n_full:
                fire_gather(j + dg, (j + dg) % nb)

        if rem:
            base = n_full * CH
            for k in range(rem // L):
                sidx_t[pl.ds(k * L, L)] = srcbuf[pl.ds(base + k * L, L)]
                didx_t[pl.ds(k * L, L)] = dstbuf[pl.ds(base + k * L, L)]
            pltpu.async_copy(table.at[sidx_t], rows_t, gsems[0]).wait()
            pltpu.sync_copy(rows_t, agg.at[didx_t], add=True)

        spt = _round_up(rpt, CH)
        ns_chunks = spt // CH
        pltpu.sync_copy(selfsrc_hbm.at[pl.ds(s * spt, spt)],
                        srcbuf.at[pl.ds(0, spt)])
        pltpu.sync_copy(selfdst_hbm.at[pl.ds(s * spt, spt)],
                        dstbuf.at[pl.ds(0, spt)])
        for g0 in range(0, ns_chunks, nb):
            grp = list(range(g0, min(g0 + nb, ns_chunks)))
            for t in grp:
                fire_gather(t, t % nb)
            for t in grp:
                finish_chunk(t, t % nb)

        plsc.subcore_barrier()
        pltpu.sync_copy(agg.at[pl.ds(s * rpt, rpt)],
                        out_hbm.at[c, pl.ds(s * rpt, rpt)])

    return scatter_kernel


def _tc1_body(x_ref, w_ref, cnt_ref, hp_ref, d_ref):
    dh = hp_ref.shape[2]
    deg = jnp.sum(cnt_ref[...], axis=1) + 1.0
    dval = lax.rsqrt(deg)
    d_ref[...] = dval[:, None]
    h = jnp.dot(x_ref[...], w_ref[...], preferred_element_type=jnp.float32)
    h = h * dval[:, None]
    hp_ref[0] = h[:, :dh]
    hp_ref[1] = h[:, dh:]


def _tc2_body(p_ref, d_ref, bg_ref, wfc_ref, bfc_ref, out_ref):
    agg = jnp.concatenate([p_ref[0], p_ref[1]], axis=1)
    aggn = agg * d_ref[...] + bg_ref[...][None, :]
    o = jnp.maximum(aggn, 0.0)
    logits = jnp.dot(o, wfc_ref[...], preferred_element_type=jnp.float32)
    logits = logits + bfc_ref[...][None, :]
    out_ref[...] = 1.0 / (1.0 + jnp.exp(-logits))


def kernel(x, edge_index, W_gcn, b_gcn, W_fc, b_fc):
    n, d_in = x.shape
    d_out = W_gcn.shape[1]
    e = edge_index.shape[1]
    dh = d_out // NC

    nb = 5
    pt_deg = e // NW
    pt = e // NS
    R = _round_up(n + 1, NS * 8)
    rpt = R // NS

    src_e = jnp.asarray(edge_index[0], jnp.int32)
    dst_e = jnp.asarray(edge_index[1], jnp.int32)
    zeros_blk = jnp.zeros((rpt, dh), jnp.float32)

    spt = _round_up(rpt, CH)
    row_i = jnp.arange(NS * spt, dtype=jnp.int32)
    node = (row_i // spt) * rpt + (row_i % spt)
    valid = ((row_i % spt) < rpt) & (node < n)
    self_src = jnp.where(valid, node, 0)
    self_dst = jnp.where(valid, node, n)

    cnt = _make_deg_kernel(pt_deg, R)(dst_e)
    cnt_t = cnt.T

    bs = 1000
    grid = n // bs
    hp, d_col = pl.pallas_call(
        _tc1_body,
        grid=(grid,),
        in_specs=[
            pl.BlockSpec((bs, d_in), lambda i: (i, 0)),
            pl.BlockSpec((d_in, d_out), lambda i: (0, 0)),
            pl.BlockSpec((bs, NW), lambda i: (i, 0)),
        ],
        out_specs=[
            pl.BlockSpec((NC, bs, dh), lambda i: (0, i, 0)),
            pl.BlockSpec((bs, 1), lambda i: (i, 0)),
        ],
        out_shape=[
            jax.ShapeDtypeStruct((NC, n, dh), jnp.float32),
            jax.ShapeDtypeStruct((n, 1), jnp.float32),
        ],
    )(x, W_gcn, cnt_t)

    partials = _make_scatter_kernel(pt, R, dh, nb)(
        hp, src_e, dst_e, self_src, self_dst, zeros_blk)

    out = pl.pallas_call(
        _tc2_body,
        grid=(grid,),
        in_specs=[
            pl.BlockSpec((NC, bs, dh), lambda i: (0, i, 0)),
            pl.BlockSpec((bs, 1), lambda i: (i, 0)),
            pl.BlockSpec((d_out,), lambda i: (0,)),
            pl.BlockSpec((d_out, 1), lambda i: (0, 0)),
            pl.BlockSpec((1,), lambda i: (0,)),
        ],
        out_specs=pl.BlockSpec((bs, 1), lambda i: (i, 0)),
        out_shape=jax.ShapeDtypeStruct((n, 1), jnp.float32),
    )(partials, d_col, b_gcn, W_fc, b_fc)

    return out


# speedup vs baseline: 1.0244x; 1.0244x over previous
import functools

import jax
import jax.numpy as jnp
from jax import lax
from jax.experimental import pallas as pl
from jax.experimental.pallas import tpu as pltpu
from jax.experimental.pallas import tpu_sc as plsc

NC = 2
NS = 16
NW = NC * NS
L = 16
CH = 128


def _round_up(a, b):
    return (a + b - 1) // b * b


def _make_deg_kernel(pt, R):
    mesh = plsc.VectorSubcoreMesh(core_axis_name="c", subcore_axis_name="s",
                                  num_cores=NC, num_subcores=NS)

    @functools.partial(
        pl.kernel,
        out_type=jax.ShapeDtypeStruct((NW, R), jnp.float32),
        mesh=mesh,
        scratch_types=[
            pltpu.VMEM((pt,), jnp.int32),
            pltpu.VMEM((R,), jnp.float32),
        ],
        compiler_params=pltpu.CompilerParams(needs_layout_passes=False),
    )
    def deg_kernel(dst_hbm, out_hbm, dstbuf, degloc):
        c = lax.axis_index("c")
        s = lax.axis_index("s")
        wid = s * NC + c
        pltpu.sync_copy(dst_hbm.at[pl.ds(wid * pt, pt)], dstbuf)

        def zbody(i, carry):
            degloc[pl.ds(i * L, L)] = jnp.zeros((L,), jnp.float32)
            return carry

        lax.fori_loop(0, R // L, zbody, 0)

        ones = jnp.ones((L,), jnp.float32)

        def cbody(j, carry):
            idx = dstbuf[pl.ds(j * L, L)]
            plsc.addupdate_scatter(degloc, [idx], ones)
            return carry

        lax.fori_loop(0, pt // L, cbody, 0)
        pltpu.sync_copy(degloc, out_hbm.at[wid])

    return deg_kernel


def _make_scatter_kernel(pt, R, dh, nb):
    mesh = plsc.VectorSubcoreMesh(core_axis_name="c", subcore_axis_name="s",
                                  num_cores=NC, num_subcores=NS)
    rpt = R // NS
    n_full = pt // CH
    rem = pt % CH
    dg = 3

    scratch = [
        pltpu.VMEM((pt,), jnp.int32),
        pltpu.VMEM((pt,), jnp.int32),
        pltpu.VMEM((CH,), jnp.int32),
    ]
    scratch += [pltpu.VMEM((CH, dh), jnp.float32)] * nb
    scratch += [pltpu.VMEM((CH,), jnp.int32)] * nb
    if rem:
        scratch += [pltpu.VMEM((rem, dh), jnp.float32),
                    pltpu.VMEM((rem,), jnp.int32),
                    pltpu.VMEM((rem,), jnp.int32)]
    scratch += [pltpu.VMEM_SHARED((R, dh), jnp.float32)]
    scratch += [pltpu.SemaphoreType.DMA] * nb

    @functools.partial(
        pl.kernel,
        out_type=jax.ShapeDtypeStruct((NC, R, dh), jnp.float32),
        mesh=mesh,
        scratch_types=scratch,
        compiler_params=pltpu.CompilerParams(needs_layout_passes=False,
                                             use_tc_tiling_on_sc=False),
    )
    def scatter_kernel(hp_hbm, src_hbm, dst_hbm, zeros_hbm, out_hbm,
                       srcbuf, dstbuf, didx, *rest):
        rows = rest[:nb]
        sidx = rest[nb:2 * nb]
        rest = rest[2 * nb:]
        if rem:
            rows_t, sidx_t, didx_t = rest[:3]
            rest = rest[3:]
        agg = rest[0]
        gsems = rest[1:]

        c = lax.axis_index("c")
        s = lax.axis_index("s")
        table = hp_hbm.at[c]
        pltpu.sync_copy(zeros_hbm, agg.at[pl.ds(s * rpt, rpt)])
        pltpu.sync_copy(src_hbm.at[pl.ds(s * pt, pt)], srcbuf)
        pltpu.sync_copy(dst_hbm.at[pl.ds(s * pt, pt)], dstbuf)
        plsc.subcore_barrier()

        def fire_gather(j, b):
            for k in range(CH // L):
                sidx[b][pl.ds(k * L, L)] = srcbuf[pl.ds(j * CH + k * L, L)]
            pltpu.async_copy(table.at[sidx[b]], rows[b], gsems[b])

        def finish_chunk(j, b):
            pltpu.make_async_copy(
                table.at[sidx[b]], rows[b], gsems[b]).wait()
            for k in range(CH // L):
                didx[pl.ds(k * L, L)] = dstbuf[pl.ds(j * CH + k * L, L)]
            pltpu.sync_copy(rows[b], agg.at[didx], add=True)

        for k in range(dg):
            fire_gather(k, k)

        def body(g, carry):
            for b in range(nb):
                j = g * nb + b
                finish_chunk(j, b)
                fire_gather(j + dg, (b + dg) % nb)
            return carry

        n_main = (n_full - nb) // nb
        lax.fori_loop(0, n_main, body, 0)
        for j in range(n_main * nb, n_full):
            b = j % nb
            finish_chunk(j, b)
            if j + dg < n_full:
                fire_gather(j + dg, (j + dg) % nb)

        if rem:
            base = n_full * CH
            for k in range(rem // L):
                sidx_t[pl.ds(k * L, L)] = srcbuf[pl.ds(base + k * L, L)]
                didx_t[pl.ds(k * L, L)] = dstbuf[pl.ds(base + k * L, L)]
            pltpu.async_copy(table.at[sidx_t], rows_t, gsems[0]).wait()
            pltpu.sync_copy(rows_t, agg.at[didx_t], add=True)

        plsc.subcore_barrier()
        pltpu.sync_copy(agg.at[pl.ds(s * rpt, rpt)],
                        out_hbm.at[c, pl.ds(s * rpt, rpt)])

    return scatter_kernel


def _tc1a_body(x_ref, w_ref, h_ref):
    h_ref[...] = jnp.dot(x_ref[...], w_ref[...],
                         preferred_element_type=jnp.float32)


def _tc1b_body(h_ref, cnt_ref, hp_ref, d_ref):
    dh = hp_ref.shape[2]
    deg = jnp.sum(cnt_ref[...], axis=1) + 1.0
    dval = lax.rsqrt(deg)
    d_ref[...] = dval[:, None]
    h = h_ref[...] * dval[:, None]
    hp_ref[0] = h[:, :dh]
    hp_ref[1] = h[:, dh:]


def _tc2_body(p_ref, hp_ref, d_ref, bg_ref, wfc_ref, bfc_ref, out_ref):
    hp = jnp.concatenate([hp_ref[0], hp_ref[1]], axis=1)
    agg = jnp.concatenate([p_ref[0], p_ref[1]], axis=1) + hp
    aggn = agg * d_ref[...] + bg_ref[...][None, :]
    o = jnp.maximum(aggn, 0.0)
    logits = jnp.dot(o, wfc_ref[...], preferred_element_type=jnp.float32)
    logits = logits + bfc_ref[...][None, :]
    out_ref[...] = 1.0 / (1.0 + jnp.exp(-logits))


def kernel(x, edge_index, W_gcn, b_gcn, W_fc, b_fc):
    n, d_in = x.shape
    d_out = W_gcn.shape[1]
    e = edge_index.shape[1]
    dh = d_out // NC

    nb = 5
    pt_deg = e // NW
    pt = e // NS
    R = _round_up(n + 1, NS * 8)
    rpt = R // NS

    src_e = jnp.asarray(edge_index[0], jnp.int32)
    dst_e = jnp.asarray(edge_index[1], jnp.int32)
    zeros_blk = jnp.zeros((rpt, dh), jnp.float32)


    cnt = _make_deg_kernel(pt_deg, R)(dst_e)
    cnt_t = cnt.T

    bs = 1000
    grid = n // bs
    h = pl.pallas_call(
        _tc1a_body,
        grid=(grid,),
        in_specs=[
            pl.BlockSpec((bs, d_in), lambda i: (i, 0)),
            pl.BlockSpec((d_in, d_out), lambda i: (0, 0)),
        ],
        out_specs=pl.BlockSpec((bs, d_out), lambda i: (i, 0)),
        out_shape=jax.ShapeDtypeStruct((n, d_out), jnp.float32),
    )(x, W_gcn)

    hp, d_col = pl.pallas_call(
        _tc1b_body,
        grid=(grid,),
        in_specs=[
            pl.BlockSpec((bs, d_out), lambda i: (i, 0)),
            pl.BlockSpec((bs, NW), lambda i: (i, 0)),
        ],
        out_specs=[
            pl.BlockSpec((NC, bs, dh), lambda i: (0, i, 0)),
            pl.BlockSpec((bs, 1), lambda i: (i, 0)),
        ],
        out_shape=[
            jax.ShapeDtypeStruct((NC, n, dh), jnp.float32),
            jax.ShapeDtypeStruct((n, 1), jnp.float32),
        ],
    )(h, cnt_t)

    partials = _make_scatter_kernel(pt, R, dh, nb)(
        hp, src_e, dst_e, zeros_blk)

    out = pl.pallas_call(
        _tc2_body,
        grid=(grid,),
        in_specs=[
            pl.BlockSpec((NC, bs, dh), lambda i: (0, i, 0)),
            pl.BlockSpec((NC, bs, dh), lambda i: (0, i, 0)),
            pl.BlockSpec((bs, 1), lambda i: (i, 0)),
            pl.BlockSpec((d_out,), lambda i: (0,)),
            pl.BlockSpec((d_out, 1), lambda i: (0, 0)),
            pl.BlockSpec((1,), lambda i: (0,)),
        ],
        out_specs=pl.BlockSpec((bs, 1), lambda i: (i, 0)),
        out_shape=jax.ShapeDtypeStruct((n, 1), jnp.float32),
    )(partials, hp, d_col, b_gcn, W_fc, b_fc)

    return out


# speedup vs baseline: 1.0256x; 1.0011x over previous
import functools

import jax
import jax.numpy as jnp
from jax import lax
from jax.experimental import pallas as pl
from jax.experimental.pallas import tpu as pltpu
from jax.experimental.pallas import tpu_sc as plsc

NC = 2
NS = 16
NW = NC * NS
L = 16
CH = 128


def _round_up(a, b):
    return (a + b - 1) // b * b


def _make_deg_kernel(pt, R):
    mesh = plsc.VectorSubcoreMesh(core_axis_name="c", subcore_axis_name="s",
                                  num_cores=NC, num_subcores=NS)

    @functools.partial(
        pl.kernel,
        out_type=jax.ShapeDtypeStruct((NW, R), jnp.float32),
        mesh=mesh,
        scratch_types=[
            pltpu.VMEM((pt,), jnp.int32),
            pltpu.VMEM((R,), jnp.float32),
        ],
        compiler_params=pltpu.CompilerParams(needs_layout_passes=False),
    )
    def deg_kernel(dst_hbm, out_hbm, dstbuf, degloc):
        c = lax.axis_index("c")
        s = lax.axis_index("s")
        wid = s * NC + c
        pltpu.sync_copy(dst_hbm.at[pl.ds(wid * pt, pt)], dstbuf)

        def zbody(i, carry):
            degloc[pl.ds(i * L, L)] = jnp.zeros((L,), jnp.float32)
            return carry

        lax.fori_loop(0, R // L, zbody, 0)

        ones = jnp.ones((L,), jnp.float32)
        unroll = 4

        def cbody(j, carry):
            for u in range(unroll):
                idx = dstbuf[pl.ds((j * unroll + u) * L, L)]
                plsc.addupdate_scatter(degloc, [idx], ones)
            return carry

        nv = pt // L
        lax.fori_loop(0, nv // unroll, cbody, 0)
        for u in range(nv - nv % unroll, nv):
            idx = dstbuf[pl.ds(u * L, L)]
            plsc.addupdate_scatter(degloc, [idx], ones)
        pltpu.sync_copy(degloc, out_hbm.at[wid])

    return deg_kernel


def _make_scatter_kernel(pt, R, dh, nb):
    mesh = plsc.VectorSubcoreMesh(core_axis_name="c", subcore_axis_name="s",
                                  num_cores=NC, num_subcores=NS)
    rpt = R // NS
    n_full = pt // CH
    rem = pt % CH
    dg = 3

    scratch = [
        pltpu.VMEM((pt,), jnp.int32),
        pltpu.VMEM((pt,), jnp.int32),
        pltpu.VMEM((CH,), jnp.int32),
    ]
    scratch += [pltpu.VMEM((CH, dh), jnp.float32)] * nb
    scratch += [pltpu.VMEM((CH,), jnp.int32)] * nb
    if rem:
        scratch += [pltpu.VMEM((rem, dh), jnp.float32),
                    pltpu.VMEM((rem,), jnp.int32),
                    pltpu.VMEM((rem,), jnp.int32)]
    scratch += [pltpu.VMEM_SHARED((R, dh), jnp.float32)]
    scratch += [pltpu.SemaphoreType.DMA] * nb

    @functools.partial(
        pl.kernel,
        out_type=jax.ShapeDtypeStruct((NC, R, dh), jnp.float32),
        mesh=mesh,
        scratch_types=scratch,
        compiler_params=pltpu.CompilerParams(needs_layout_passes=False,
                                             use_tc_tiling_on_sc=False),
    )
    def scatter_kernel(hp_hbm, src_hbm, dst_hbm, zeros_hbm, out_hbm,
                       srcbuf, dstbuf, didx, *rest):
        rows = rest[:nb]
        sidx = rest[nb:2 * nb]
        rest = rest[2 * nb:]
        if rem:
            rows_t, sidx_t, didx_t = rest[:3]
            rest = rest[3:]
        agg = rest[0]
        gsems = rest[1:]

        c = lax.axis_index("c")
        s = lax.axis_index("s")
        table = hp_hbm.at[c]
        pltpu.sync_copy(zeros_hbm, agg.at[pl.ds(s * rpt, rpt)])
        pltpu.sync_copy(src_hbm.at[pl.ds(s * pt, pt)], srcbuf)
        pltpu.sync_copy(dst_hbm.at[pl.ds(s * pt, pt)], dstbuf)
        plsc.subcore_barrier()

        def fire_gather(j, b):
            for k in range(CH // L):
                sidx[b][pl.ds(k * L, L)] = srcbuf[pl.ds(j * CH + k * L, L)]
            pltpu.async_copy(table.at[sidx[b]], rows[b], gsems[b])

        def finish_chunk(j, b):
            pltpu.make_async_copy(
                table.at[sidx[b]], rows[b], gsems[b]).wait()
            for k in range(CH // L):
                didx[pl.ds(k * L, L)] = dstbuf[pl.ds(j * CH + k * L, L)]
            pltpu.sync_copy(rows[b], agg.at[didx], add=True)

        for k in range(dg):
            fire_gather(k, k)

        def body(g, carry):
            for b in range(nb):
                j = g * nb + b
                finish_chunk(j, b)
                fire_gather(j + dg, (b + dg) % nb)
            return carry

        n_main = (n_full - nb) // nb
        lax.fori_loop(0, n_main, body, 0)
        for j in range(n_main * nb, n_full):
            b = j % nb
            finish_chunk(j, b)
            if j + dg < n_full:
                fire_gather(j + dg, (j + dg) % nb)

        if rem:
            base = n_full * CH
            for k in range(rem // L):
                sidx_t[pl.ds(k * L, L)] = srcbuf[pl.ds(base + k * L, L)]
                didx_t[pl.ds(k * L, L)] = dstbuf[pl.ds(base + k * L, L)]
            pltpu.async_copy(table.at[sidx_t], rows_t, gsems[0]).wait()
            pltpu.sync_copy(rows_t, agg.at[didx_t], add=True)

        plsc.subcore_barrier()
        pltpu.sync_copy(agg.at[pl.ds(s * rpt, rpt)],
                        out_hbm.at[c, pl.ds(s * rpt, rpt)])

    return scatter_kernel


def _tc1a_body(x_ref, w_ref, h_ref):
    h_ref[...] = jnp.dot(x_ref[...], w_ref[...],
                         preferred_element_type=jnp.float32)


def _tc1b_body(h_ref, cnt_ref, hp_ref, d_ref):
    dh = hp_ref.shape[2]
    deg = jnp.sum(cnt_ref[...], axis=1) + 1.0
    dval = lax.rsqrt(deg)
    d_ref[...] = dval[:, None]
    h = h_ref[...] * dval[:, None]
    hp_ref[0] = h[:, :dh]
    hp_ref[1] = h[:, dh:]


def _tc2_body(p_ref, hp_ref, d_ref, bg_ref, wfc_ref, bfc_ref, out_ref):
    hp = jnp.concatenate([hp_ref[0], hp_ref[1]], axis=1)
    agg = jnp.concatenate([p_ref[0], p_ref[1]], axis=1) + hp
    aggn = agg * d_ref[...] + bg_ref[...][None, :]
    o = jnp.maximum(aggn, 0.0)
    logits = jnp.dot(o, wfc_ref[...], preferred_element_type=jnp.float32)
    logits = logits + bfc_ref[...][None, :]
    out_ref[...] = 1.0 / (1.0 + jnp.exp(-logits))


def kernel(x, edge_index, W_gcn, b_gcn, W_fc, b_fc):
    n, d_in = x.shape
    d_out = W_gcn.shape[1]
    e = edge_index.shape[1]
    dh = d_out // NC

    nb = 5
    pt_deg = e // NW
    pt = e // NS
    R = _round_up(n + 1, NS * 8)
    rpt = R // NS

    src_e = jnp.asarray(edge_index[0], jnp.int32)
    dst_e = jnp.asarray(edge_index[1], jnp.int32)
    zeros_blk = jnp.zeros((rpt, dh), jnp.float32)


    cnt = _make_deg_kernel(pt_deg, R)(dst_e)
    cnt_t = cnt.T

    bs = 1000
    grid = n // bs
    h = pl.pallas_call(
        _tc1a_body,
        grid=(grid,),
        in_specs=[
            pl.BlockSpec((bs, d_in), lambda i: (i, 0)),
            pl.BlockSpec((d_in, d_out), lambda i: (0, 0)),
        ],
        out_specs=pl.BlockSpec((bs, d_out), lambda i: (i, 0)),
        out_shape=jax.ShapeDtypeStruct((n, d_out), jnp.float32),
    )(x, W_gcn)

    hp, d_col = pl.pallas_call(
        _tc1b_body,
        grid=(grid,),
        in_specs=[
            pl.BlockSpec((bs, d_out), lambda i: (i, 0)),
            pl.BlockSpec((bs, NW), lambda i: (i, 0)),
        ],
        out_specs=[
            pl.BlockSpec((NC, bs, dh), lambda i: (0, i, 0)),
            pl.BlockSpec((bs, 1), lambda i: (i, 0)),
        ],
        out_shape=[
            jax.ShapeDtypeStruct((NC, n, dh), jnp.float32),
            jax.ShapeDtypeStruct((n, 1), jnp.float32),
        ],
    )(h, cnt_t)

    partials = _make_scatter_kernel(pt, R, dh, nb)(
        hp, src_e, dst_e, zeros_blk)

    out = pl.pallas_call(
        _tc2_body,
        grid=(grid,),
        in_specs=[
            pl.BlockSpec((NC, bs, dh), lambda i: (0, i, 0)),
            pl.BlockSpec((NC, bs, dh), lambda i: (0, i, 0)),
            pl.BlockSpec((bs, 1), lambda i: (i, 0)),
            pl.BlockSpec((d_out,), lambda i: (0,)),
            pl.BlockSpec((d_out, 1), lambda i: (0, 0)),
            pl.BlockSpec((1,), lambda i: (0,)),
        ],
        out_specs=pl.BlockSpec((bs, 1), lambda i: (i, 0)),
        out_shape=jax.ShapeDtypeStruct((n, 1), jnp.float32),
    )(partials, hp, d_col, b_gcn, W_fc, b_fc)

    return out
